# baseline (speedup 1.0000x reference)
import functools
import math

import jax
import jax.numpy as jnp
from jax import lax
from jax.experimental import pallas as pl
from jax.experimental.pallas import tpu as pltpu
from jax.experimental.pallas import tpu_sc as plsc

N = 10000
E = 320000
D = 128
G = 64
MID = 64

NC = 2
NS = 16
NW = NC * NS
EW = E // NW
CH = 80
NCHUNK = EW // CH
RB = 80
NRC = N // RB
KMAX = NRC // NS + 1

BLK = 1000
NBLK = N // BLK

_BN_SCALE = 1.0 / math.sqrt(1.0 + 1e-5)


def _sc_gather_scatter(hb, pk, at3, wvec):
    mesh = plsc.VectorSubcoreMesh(core_axis_name="c", subcore_axis_name="s")

    @functools.partial(
        pl.kernel,
        out_type=jax.ShapeDtypeStruct((NC, N, D), jnp.float32),
        mesh=mesh,
        scratch_types=[
            [pltpu.VMEM((2, CH), jnp.int32) for _ in range(4)],
            [pltpu.VMEM((CH,), jnp.float32) for _ in range(4)],
            [pltpu.VMEM((CH, D), jnp.float32) for _ in range(2)],
            pltpu.VMEM((D,), jnp.float32),
            pltpu.VMEM_SHARED((N, D), jnp.float32),
            [pltpu.SemaphoreType.DMA for _ in range(4)],
            [pltpu.SemaphoreType.DMA for _ in range(2)],
            [pltpu.SemaphoreType.DMA for _ in range(2)],
        ],
    )
    def k(hb_hbm, pk_hbm, at_hbm, w_hbm, out_hbm,
          idxb, atb, rows, w_v, acc_sh, sem_i, sem_g, sem_s):
        c = lax.axis_index("c")
        s = lax.axis_index("s")
        wid = c * NS + s
        rows0 = rows[0]

        def load_idx(i, b):
            pltpu.async_copy(pk_hbm.at[wid, i], idxb[b], sem_i[b])
            pltpu.async_copy(at_hbm.at[wid, i], atb[b], sem_i[b])

        def wait_idx(i, b):
            pltpu.make_async_copy(pk_hbm.at[wid, i], idxb[b],
                                  sem_i[b]).wait()
            pltpu.make_async_copy(at_hbm.at[wid, i], atb[b],
                                  sem_i[b]).wait()

        def gather(i4, b):
            pltpu.async_copy(hb_hbm.at[idxb[i4].at[0]], rows[b], sem_g[b])

        def wait_gather(i4, b):
            pltpu.make_async_copy(hb_hbm.at[idxb[i4].at[0]], rows[b],
                                  sem_g[b]).wait()

        def scatter(i4, b):
            pltpu.async_copy(rows[b], acc_sh.at[idxb[i4].at[1]], sem_s[b],
                             add=True)

        def wait_scatter(i4, b):
            pltpu.make_async_copy(rows[b], acc_sh.at[idxb[i4].at[1]],
                                  sem_s[b]).wait()

        load_idx(0, 0)
        pltpu.sync_copy(w_hbm, w_v)

        zeros16 = jnp.zeros((16,), jnp.float32)

        def zb(j, carry):
            for g in range(D // 16):
                rows0[j, pl.ds(g * 16, 16)] = zeros16
            return carry

        lax.fori_loop(0, RB, zb, 0)
        for kk in range(KMAX):
            chunk = s + kk * NS

            @pl.when(chunk < NRC)
            def _():
                pltpu.sync_copy(rows0, acc_sh.at[pl.ds(chunk * RB, RB)])

        wait_idx(0, 0)
        plsc.subcore_barrier()

        w_regs = [w_v[pl.ds(g * 16, 16)] for g in range(D // 16)]

        def compute(i4, b):
            rv = rows[b]
            ab = atb[i4]

            def edge16(j16, c2):
                av = ab[pl.ds(j16 * 16, 16)]
                j0 = j16 * 16
                for kk in range(16):
                    a = av[kk]
                    for g in range(D // 16):
                        sl = pl.ds(g * 16, 16)
                        rv[j0 + kk, sl] = jnp.maximum(
                            rv[j0 + kk, sl] + a * w_regs[g], 0.0)
                return c2

            lax.fori_loop(0, CH // 16, edge16, 0)

        gather(0, 0)
        load_idx(1, 1)

        @pl.loop(0, NCHUNK - 1, step=4)
        def _(t):
            for b in range(4):
                i = t + b
                i4 = b
                n4 = (b + 1) % 4
                r = b % 2
                nr = 1 - r
                wait_idx(i + 1, n4)

                if b == 0:
                    @pl.when(t > 0)
                    def _():
                        wait_scatter(3, nr)
                else:
                    wait_scatter(b - 1, nr)

                compute(i4, r)
                scatter(i4, r)

                @pl.when(i + 2 < NCHUNK)
                def _():
                    load_idx(i + 2, (b + 2) % 4)

        wait_scatter(3, 1)
        compute(0, 0)
        scatter(0, 0)
        wait_scatter(0, 0)
        plsc.subcore_barrier()

        for kk in range(KMAX):
            chunk = s + kk * NS

            @pl.when(chunk < NRC)
            def _():
                r0 = chunk * RB
                pltpu.sync_copy(acc_sh.at[pl.ds(r0, RB)], rows0)
                pltpu.sync_copy(rows0, out_hbm.at[c, pl.ds(r0, RB)])

    return k(hb, pk, at3, wvec)


def _tc_encoder(x, W, bias):
    def body(x_ref, w_ref, b_ref, o_ref):
        o_ref[...] = jnp.dot(
            x_ref[...], w_ref[...], preferred_element_type=jnp.float32
        ) + b_ref[...]

    return pl.pallas_call(
        body,
        grid=(NBLK,),
        in_specs=[
            pl.BlockSpec((BLK, D), lambda i: (i, 0)),
            pl.BlockSpec((D, D), lambda i: (0, 0)),
            pl.BlockSpec((1, D), lambda i: (0, 0)),
        ],
        out_specs=pl.BlockSpec((BLK, D), lambda i: (i, 0)),
        out_shape=jax.ShapeDtypeStruct((N, D), jnp.float32),
    )(x, W, bias.reshape(1, D))


def _tc_layer(hb, parts, b_edge, eps, W1, b1, W2, b2, g, be):
    def body(eps_ref, hb_ref, p_ref, bed_ref, w1_ref, b1_ref, w2_ref, b2_ref,
             gs_ref, be_ref, o_ref):
        h = hb_ref[...] - bed_ref[...]
        agg = p_ref[0] + p_ref[1]
        z = (1.0 + eps_ref[0]) * h + agg
        z = jnp.maximum(
            jnp.dot(z, w1_ref[...], preferred_element_type=jnp.float32)
            + b1_ref[...], 0.0)
        z = jnp.dot(z, w2_ref[...], preferred_element_type=jnp.float32) \
            + b2_ref[...]
        z = z * gs_ref[...] + be_ref[...]
        o_ref[...] = jnp.maximum(z, 0.0) + bed_ref[...]

    return pl.pallas_call(
        body,
        grid=(NBLK,),
        in_specs=[
            pl.BlockSpec(memory_space=pltpu.SMEM),
            pl.BlockSpec((BLK, D), lambda i: (i, 0)),
            pl.BlockSpec((2, BLK, D), lambda i: (0, i, 0)),
            pl.BlockSpec((1, D), lambda i: (0, 0)),
            pl.BlockSpec((D, D), lambda i: (0, 0)),
            pl.BlockSpec((1, D), lambda i: (0, 0)),
            pl.BlockSpec((D, D), lambda i: (0, 0)),
            pl.BlockSpec((1, D), lambda i: (0, 0)),
            pl.BlockSpec((1, D), lambda i: (0, 0)),
            pl.BlockSpec((1, D), lambda i: (0, 0)),
        ],
        out_specs=pl.BlockSpec((BLK, D), lambda i: (i, 0)),
        out_shape=jax.ShapeDtypeStruct((N, D), jnp.float32),
    )(jnp.reshape(eps, (1,)), hb, parts, b_edge.reshape(1, D), W1,
      b1.reshape(1, D), W2, b2.reshape(1, D),
      (g * _BN_SCALE).reshape(1, D), be.reshape(1, D))


def _tc_final(hb, parts, b_edge, eps, W1, b1, W2, b2, g, be, batch,
              Wc1, bc1, Wc2, bc2):
    def body(eps_ref, hb_ref, p_ref, bed_ref, w1_ref, b1_ref, w2_ref, b2_ref,
             gs_ref, be_ref, bat_ref, wc1_ref, bc1_ref, wc2_ref, bc2_ref,
             o_ref, pool_acc, cnt_acc):
        i = pl.program_id(0)
        h = hb_ref[...] - bed_ref[...]
        agg = p_ref[0] + p_ref[1]
        z = (1.0 + eps_ref[0]) * h + agg
        z = jnp.maximum(
            jnp.dot(z, w1_ref[...], preferred_element_type=jnp.float32)
            + b1_ref[...], 0.0)
        z = jnp.dot(z, w2_ref[...], preferred_element_type=jnp.float32) \
            + b2_ref[...]
        h2 = jnp.maximum(z * gs_ref[...] + be_ref[...], 0.0)

        mask = (bat_ref[...] ==
                lax.broadcasted_iota(jnp.int32, (BLK, G), 1)
                ).astype(jnp.float32)
        pb = lax.dot_general(mask, h2, (((0,), (0,)), ((), ())),
                             preferred_element_type=jnp.float32)
        cb = lax.dot_general(mask, jnp.ones((BLK, 128), jnp.float32),
                             (((0,), (0,)), ((), ())),
                             preferred_element_type=jnp.float32)

        @pl.when(i == 0)
        def _():
            pool_acc[...] = pb
            cnt_acc[...] = cb

        @pl.when(i > 0)
        def _():
            pool_acc[...] += pb
            cnt_acc[...] += cb

        @pl.when(i == NBLK - 1)
        def _():
            pooled = pool_acc[...] / jnp.maximum(cnt_acc[...], 1.0)
            zz = jnp.maximum(
                jnp.dot(pooled, wc1_ref[...],
                        preferred_element_type=jnp.float32)
                + bc1_ref[...], 0.0)
            oo = jnp.dot(zz, wc2_ref[...],
                         preferred_element_type=jnp.float32) + bc2_ref[...]
            o_ref[...] = 1.0 / (1.0 + jnp.exp(-oo))

    return pl.pallas_call(
        body,
        grid=(NBLK,),
        in_specs=[
            pl.BlockSpec(memory_space=pltpu.SMEM),
            pl.BlockSpec((BLK, D), lambda i: (i, 0)),
            pl.BlockSpec((2, BLK, D), lambda i: (0, i, 0)),
            pl.BlockSpec((1, D), lambda i: (0, 0)),
            pl.BlockSpec((D, D), lambda i: (0, 0)),
            pl.BlockSpec((1, D), lambda i: (0, 0)),
            pl.BlockSpec((D, D), lambda i: (0, 0)),
            pl.BlockSpec((1, D), lambda i: (0, 0)),
            pl.BlockSpec((1, D), lambda i: (0, 0)),
            pl.BlockSpec((1, D), lambda i: (0, 0)),
            pl.BlockSpec((BLK, 1), lambda i: (i, 0)),
            pl.BlockSpec((D, MID), lambda i: (0, 0)),
            pl.BlockSpec((1, MID), lambda i: (0, 0)),
            pl.BlockSpec((MID, 1), lambda i: (0, 0)),
            pl.BlockSpec((1, 1), lambda i: (0, 0)),
        ],
        out_specs=pl.BlockSpec((G, 1), lambda i: (0, 0)),
        out_shape=jax.ShapeDtypeStruct((G, 1), jnp.float32),
        scratch_shapes=[
            pltpu.VMEM((G, D), jnp.float32),
            pltpu.VMEM((G, 128), jnp.float32),
        ],
    )(jnp.reshape(eps, (1,)), hb, parts, b_edge.reshape(1, D), W1,
      b1.reshape(1, D), W2, b2.reshape(1, D),
      (g * _BN_SCALE).reshape(1, D), be.reshape(1, D),
      batch.reshape(N, 1), Wc1, bc1.reshape(1, MID), Wc2,
      bc2.reshape(1, 1))


def kernel(x, edge_index, edge_attr, batch, W_node, b_node, W_edge, b_edge,
           eps0, W1_0, b1_0, W2_0, b2_0, g0, be0,
           eps1, W1_1, b1_1, W2_1, b2_1, g1, be1, Wc1, bc1, Wc2, bc2):
    pk = jnp.stack(
        [edge_index[0].reshape(NW, NCHUNK, CH),
         edge_index[1].reshape(NW, NCHUNK, CH)], axis=2)
    at3 = edge_attr[:, 0].reshape(NW, NCHUNK, CH)
    wvec = W_edge[0]

    hb0 = _tc_encoder(x, W_node, b_node + b_edge)
    parts0 = _sc_gather_scatter(hb0, pk, at3, wvec)
    hb1 = _tc_layer(hb0, parts0, b_edge, eps0, W1_0, b1_0, W2_0, b2_0,
                    g0, be0)
    parts1 = _sc_gather_scatter(hb1, pk, at3, wvec)
    out2d = _tc_final(hb1, parts1, b_edge, eps1, W1_1, b1_1, W2_1, b2_1,
                      g1, be1, batch, Wc1, bc1, Wc2, bc2)
    return out2d[:, 0]


# speedup vs baseline: 1.0887x; 1.0887x over previous
import functools
import math

import jax
import jax.numpy as jnp
from jax import lax
from jax.experimental import pallas as pl
from jax.experimental.pallas import tpu as pltpu
from jax.experimental.pallas import tpu_sc as plsc

N = 10000
E = 320000
D = 128
G = 64
MID = 64

NC = 2
NS = 16
NW = NC * NS
EW = E // NW
CH = 80
NCHUNK = EW // CH
RB = 80
NRC = N // RB
KMAX = NRC // NS + 1

BLK = 1000
NBLK = N // BLK

_BN_SCALE = float(1.0 / math.sqrt(1.0 + 1e-5))


def _sc_gather_scatter(hb, pk, at3, wvec):
    mesh = plsc.VectorSubcoreMesh(core_axis_name="c", subcore_axis_name="s")

    @functools.partial(
        pl.kernel,
        out_type=jax.ShapeDtypeStruct((NC, N, D), jnp.float32),
        mesh=mesh,
        scratch_types=[
            [pltpu.VMEM((2, CH), jnp.int32) for _ in range(2)],
            [pltpu.VMEM((CH,), jnp.float32) for _ in range(2)],
            [pltpu.VMEM((CH, D), jnp.float32) for _ in range(2)],
            [pltpu.VMEM((CH,), jnp.int32) for _ in range(2)],
            pltpu.VMEM((D,), jnp.float32),
            pltpu.VMEM_SHARED((N, D), jnp.float32),
            [pltpu.SemaphoreType.DMA for _ in range(2)],
            [pltpu.SemaphoreType.DMA for _ in range(2)],
            [pltpu.SemaphoreType.DMA for _ in range(2)],
        ],
    )
    def k(hb_hbm, pk_hbm, at_hbm, w_hbm, out_hbm,
          idxb, atb, rows, dst_s, w_v, acc_sh, sem_i, sem_g, sem_s):
        c = lax.axis_index("c")
        s = lax.axis_index("s")
        wid = c * NS + s
        rows0 = rows[0]

        def load_idx(i, b):
            pltpu.async_copy(pk_hbm.at[wid, i], idxb[b], sem_i[b])
            pltpu.async_copy(at_hbm.at[wid, i], atb[b], sem_i[b])

        def wait_idx(i, b):
            pltpu.make_async_copy(pk_hbm.at[wid, i], idxb[b],
                                  sem_i[b]).wait()
            pltpu.make_async_copy(at_hbm.at[wid, i], atb[b],
                                  sem_i[b]).wait()

        def gather(i4, b):
            pltpu.async_copy(hb_hbm.at[idxb[i4].at[0]], rows[b], sem_g[b])

        def wait_gather(i4, b):
            pltpu.make_async_copy(hb_hbm.at[idxb[i4].at[0]], rows[b],
                                  sem_g[b]).wait()

        def scatter(b):
            pltpu.async_copy(rows[b], acc_sh.at[dst_s[b]], sem_s[b],
                             add=True)

        def wait_scatter(b):
            pltpu.make_async_copy(rows[b], acc_sh.at[dst_s[b]],
                                  sem_s[b]).wait()

        load_idx(0, 0)
        pltpu.sync_copy(w_hbm, w_v)

        zeros16 = jnp.zeros((16,), jnp.float32)
        for j in range(RB):
            for g in range(D // 16):
                rows0[j, pl.ds(g * 16, 16)] = zeros16
        for kk in range(KMAX):
            chunk = s + kk * NS

            @pl.when(chunk < NRC)
            def _():
                pltpu.sync_copy(rows0, acc_sh.at[pl.ds(chunk * RB, RB)])

        wait_idx(0, 0)
        plsc.subcore_barrier()

        w_regs = [w_v[pl.ds(g * 16, 16)] for g in range(D // 16)]

        def compute(b):
            rv = rows[b]
            ab = atb[b]
            ib = idxb[b]
            ds = dst_s[b]
            for q in range(CH // 16):
                ds[pl.ds(q * 16, 16)] = ib[1, pl.ds(q * 16, 16)]
            for q in range(CH // 16):
                av = ab[pl.ds(q * 16, 16)]
                for kk in range(16):
                    a = av[kk]
                    j = q * 16 + kk
                    for g in range(D // 16):
                        sl = pl.ds(g * 16, 16)
                        rv[j, sl] = jnp.maximum(
                            rv[j, sl] + a * w_regs[g], 0.0)

        gather(0, 0)
        load_idx(1, 1)

        @pl.loop(0, NCHUNK - 1, step=2)
        def _(t):
            for b in range(2):
                i = t + b
                nb = 1 - b
                wait_idx(i + 1, nb)

                if b == 0:
                    @pl.when(t > 0)
                    def _():
                        wait_scatter(nb)
                else:
                    wait_scatter(nb)

                gather(nb, nb)
                wait_gather(b, b)
                compute(b)
                scatter(b)

                @pl.when(i + 2 < NCHUNK)
                def _():
                    load_idx(i + 2, b)

        wait_scatter(1)
        wait_gather(0, 0)
        compute(0)
        scatter(0)
        wait_scatter(0)
        plsc.subcore_barrier()

        for kk in range(KMAX):
            chunk = s + kk * NS

            @pl.when(chunk < NRC)
            def _():
                r0 = chunk * RB
                pltpu.sync_copy(acc_sh.at[pl.ds(r0, RB)], rows0)
                pltpu.sync_copy(rows0, out_hbm.at[c, pl.ds(r0, RB)])

    return k(hb, pk, at3, wvec)


def _tc_encoder(x, W, bias):
    def body(x_ref, w_ref, b_ref, o_ref):
        o_ref[...] = jnp.dot(
            x_ref[...], w_ref[...], preferred_element_type=jnp.float32
        ) + b_ref[...]

    return pl.pallas_call(
        body,
        grid=(NBLK,),
        in_specs=[
            pl.BlockSpec((BLK, D), lambda i: (i, 0)),
            pl.BlockSpec((D, D), lambda i: (0, 0)),
            pl.BlockSpec((1, D), lambda i: (0, 0)),
        ],
        out_specs=pl.BlockSpec((BLK, D), lambda i: (i, 0)),
        out_shape=jax.ShapeDtypeStruct((N, D), jnp.float32),
    )(x, W, bias.reshape(1, D))


def _tc_layer(hb, parts, b_edge, eps, W1, b1, W2, b2, g, be):
    def body(eps_ref, hb_ref, p_ref, bed_ref, w1_ref, b1_ref, w2_ref, b2_ref,
             gs_ref, be_ref, o_ref):
        h = hb_ref[...] - bed_ref[...]
        agg = p_ref[0].astype(jnp.float32) + p_ref[1].astype(jnp.float32)
        z = (1.0 + eps_ref[0]) * h + agg
        z = jnp.maximum(
            jnp.dot(z, w1_ref[...], preferred_element_type=jnp.float32)
            + b1_ref[...], 0.0)
        z = jnp.dot(z, w2_ref[...], preferred_element_type=jnp.float32) \
            + b2_ref[...]
        z = z * _BN_SCALE * gs_ref[...] + be_ref[...]
        o_ref[...] = jnp.maximum(z, 0.0) + bed_ref[...]

    return pl.pallas_call(
        body,
        grid=(NBLK,),
        in_specs=[
            pl.BlockSpec(memory_space=pltpu.SMEM),
            pl.BlockSpec((BLK, D), lambda i: (i, 0)),
            pl.BlockSpec((2, BLK, D), lambda i: (0, i, 0)),
            pl.BlockSpec((1, D), lambda i: (0, 0)),
            pl.BlockSpec((D, D), lambda i: (0, 0)),
            pl.BlockSpec((1, D), lambda i: (0, 0)),
            pl.BlockSpec((D, D), lambda i: (0, 0)),
            pl.BlockSpec((1, D), lambda i: (0, 0)),
            pl.BlockSpec((1, D), lambda i: (0, 0)),
            pl.BlockSpec((1, D), lambda i: (0, 0)),
        ],
        out_specs=pl.BlockSpec((BLK, D), lambda i: (i, 0)),
        out_shape=jax.ShapeDtypeStruct((N, D), jnp.float32),
    )(jnp.reshape(eps, (1,)), hb, parts, b_edge.reshape(1, D), W1,
      b1.reshape(1, D), W2, b2.reshape(1, D),
      g.reshape(1, D), be.reshape(1, D))


def _tc_final(hb, parts, b_edge, eps, W1, b1, W2, b2, g, be, batch,
              Wc1, bc1, Wc2, bc2):
    def body(eps_ref, hb_ref, p_ref, bed_ref, w1_ref, b1_ref, w2_ref, b2_ref,
             gs_ref, be_ref, bat_ref, wc1_ref, bc1_ref, wc2_ref, bc2_ref,
             o_ref, pool_acc, cnt_acc):
        i = pl.program_id(0)
        h = hb_ref[...] - bed_ref[...]
        agg = p_ref[0].astype(jnp.float32) + p_ref[1].astype(jnp.float32)
        z = (1.0 + eps_ref[0]) * h + agg
        z = jnp.maximum(
            jnp.dot(z, w1_ref[...], preferred_element_type=jnp.float32)
            + b1_ref[...], 0.0)
        z = jnp.dot(z, w2_ref[...], preferred_element_type=jnp.float32) \
            + b2_ref[...]
        h2 = jnp.maximum(z * _BN_SCALE * gs_ref[...] + be_ref[...], 0.0)

        mask = (bat_ref[...] ==
                lax.broadcasted_iota(jnp.int32, (BLK, G), 1)
                ).astype(jnp.float32)
        pb = lax.dot_general(mask, h2, (((0,), (0,)), ((), ())),
                             preferred_element_type=jnp.float32,
                             precision=lax.Precision.HIGHEST)
        cb = lax.dot_general(mask, jnp.ones((BLK, 128), jnp.float32),
                             (((0,), (0,)), ((), ())),
                             preferred_element_type=jnp.float32,
                             precision=lax.Precision.HIGHEST)

        @pl.when(i == 0)
        def _():
            pool_acc[...] = pb
            cnt_acc[...] = cb

        @pl.when(i > 0)
        def _():
            pool_acc[...] += pb
            cnt_acc[...] += cb

        @pl.when(i == NBLK - 1)
        def _():
            pooled = pool_acc[...] / jnp.maximum(cnt_acc[...], 1.0)
            zz = jnp.maximum(
                jnp.dot(pooled, wc1_ref[...],
                        preferred_element_type=jnp.float32)
                + bc1_ref[...], 0.0)
            oo = jnp.dot(zz, wc2_ref[...],
                         preferred_element_type=jnp.float32) + bc2_ref[...]
            o_ref[...] = 1.0 / (1.0 + jnp.exp(-oo))

    return pl.pallas_call(
        body,
        grid=(NBLK,),
        in_specs=[
            pl.BlockSpec(memory_space=pltpu.SMEM),
            pl.BlockSpec((BLK, D), lambda i: (i, 0)),
            pl.BlockSpec((2, BLK, D), lambda i: (0, i, 0)),
            pl.BlockSpec((1, D), lambda i: (0, 0)),
            pl.BlockSpec((D, D), lambda i: (0, 0)),
            pl.BlockSpec((1, D), lambda i: (0, 0)),
            pl.BlockSpec((D, D), lambda i: (0, 0)),
            pl.BlockSpec((1, D), lambda i: (0, 0)),
            pl.BlockSpec((1, D), lambda i: (0, 0)),
            pl.BlockSpec((1, D), lambda i: (0, 0)),
            pl.BlockSpec((BLK, 1), lambda i: (i, 0)),
            pl.BlockSpec((D, MID), lambda i: (0, 0)),
            pl.BlockSpec((1, MID), lambda i: (0, 0)),
            pl.BlockSpec((MID, 1), lambda i: (0, 0)),
            pl.BlockSpec((1, 1), lambda i: (0, 0)),
        ],
        out_specs=pl.BlockSpec((G, 1), lambda i: (0, 0)),
        out_shape=jax.ShapeDtypeStruct((G, 1), jnp.float32),
        scratch_shapes=[
            pltpu.VMEM((G, D), jnp.float32),
            pltpu.VMEM((G, 128), jnp.float32),
        ],
    )(jnp.reshape(eps, (1,)), hb, parts, b_edge.reshape(1, D), W1,
      b1.reshape(1, D), W2, b2.reshape(1, D),
      g.reshape(1, D), be.reshape(1, D),
      batch.reshape(N, 1), Wc1, bc1.reshape(1, MID), Wc2,
      bc2.reshape(1, 1))


def kernel(x, edge_index, edge_attr, batch, W_node, b_node, W_edge, b_edge,
           eps0, W1_0, b1_0, W2_0, b2_0, g0, be0,
           eps1, W1_1, b1_1, W2_1, b2_1, g1, be1, Wc1, bc1, Wc2, bc2):
    pk = jnp.stack(
        [edge_index[0].reshape(NW, NCHUNK, CH),
         edge_index[1].reshape(NW, NCHUNK, CH)], axis=2)
    at3 = edge_attr[:, 0].reshape(NW, NCHUNK, CH)
    wvec = W_edge[0]

    hb0 = _tc_encoder(x, W_node, b_node + b_edge)
    parts0 = _sc_gather_scatter(hb0, pk, at3, wvec)
    hb1 = _tc_layer(hb0, parts0, b_edge, eps0, W1_0, b1_0, W2_0, b2_0,
                    g0, be0)
    parts1 = _sc_gather_scatter(hb1, pk, at3, wvec)
    out2d = _tc_final(hb1, parts1, b_edge, eps1, W1_1, b1_1, W2_1, b2_1,
                      g1, be1, batch, Wc1, bc1, Wc2, bc2)
    return out2d[:, 0]
